# Initial kernel scaffold; baseline (speedup 1.0000x reference)
import jax
import jax.numpy as jnp
from jax.experimental import pallas as pl


def kernel(feat, segment_ids):
    raise NotImplementedError("write your pallas kernel here")


# speedup vs baseline: 4.5908x; 4.5908x over previous
import functools

import jax
import jax.numpy as jnp
from jax import lax
from jax.experimental import pallas as pl
from jax.experimental.pallas import tpu as pltpu
from jax.experimental.pallas import tpu_sc as plsc

N = 320000
D = 128
S = 256
NC = 2
NS = 16
NW = NC * NS
BLK = 128
NB = N // BLK


def _sc_segment_sum(feat, ids3d):
    mesh = plsc.VectorSubcoreMesh(
        core_axis_name="c", subcore_axis_name="s", num_cores=NC, num_subcores=NS
    )

    @functools.partial(
        pl.kernel,
        out_type=jax.ShapeDtypeStruct((NC, S, D), jnp.float32),
        mesh=mesh,
        scratch_types=[
            pltpu.VMEM((1, BLK), jnp.int32),
            pltpu.VMEM((BLK, D), jnp.float32),
            pltpu.VMEM((NS, D), jnp.float32),
            pltpu.VMEM_SHARED((S, D), jnp.float32),
        ],
    )
    def seg_sum(feat_hbm, ids_hbm, out_hbm, ids_v, fbuf, stage, acc):
        cid = lax.axis_index("c")
        sid = lax.axis_index("s")
        wid = cid * NS + sid

        zero = jnp.zeros((16,), jnp.float32)
        for i in range(NS):
            for j in range(D // 16):
                stage[i, pl.ds(j * 16, 16)] = zero
        pltpu.sync_copy(stage, acc.at[pl.ds(sid * NS, NS)])
        plsc.subcore_barrier()

        @pl.loop(wid, NB, step=NW)
        def _blocks(b):
            pltpu.sync_copy(ids_hbm.at[b], ids_v)
            pltpu.sync_copy(feat_hbm.at[pl.ds(b * BLK, BLK)], fbuf)
            pltpu.sync_copy(fbuf, acc.at[ids_v.at[0]], add=True)

        plsc.subcore_barrier()
        pltpu.sync_copy(acc.at[pl.ds(sid * NS, NS)], stage)
        pltpu.sync_copy(stage, out_hbm.at[cid, pl.ds(sid * NS, NS)])

    return seg_sum(feat, ids3d)


def _combine(parts_ref, o_ref):
    o_ref[...] = parts_ref[0] + parts_ref[1]


def kernel(feat, segment_ids):
    ids3d = segment_ids.astype(jnp.int32).reshape(NB, 1, BLK)
    partials = _sc_segment_sum(feat, ids3d)
    return pl.pallas_call(
        _combine,
        out_shape=jax.ShapeDtypeStruct((S, D), jnp.float32),
    )(partials)
